# baseline (speedup 1.0000x reference)
import jax
import jax.numpy as jnp
from jax.experimental import pallas as pl
from jax.experimental.pallas import tpu as pltpu

B, S, D, E, K = 4, 2048, 2048, 16, 2
NOISY_STD = 1.0
T = 2048


def _gate_body(x_ref, wt_ref, nw_ref, noise_ref, w_out_ref, idx_out_ref):
    logits_tn = jax.lax.dot_general(
        x_ref[...], wt_ref[...],
        (((1,), (0,)), ((), ())),
        preferred_element_type=jnp.float32,
    )
    lt = jnp.transpose(logits_tn)
    lt = lt + jnp.transpose(noise_ref[...]) * (NOISY_STD * nw_ref[...])

    iota = jax.lax.broadcasted_iota(jnp.int32, (E, T), 0)
    neg_inf = jnp.float32(-jnp.inf)

    m1 = jnp.max(lt, axis=0, keepdims=True)
    idx1 = jnp.min(jnp.where(lt == m1, iota, E), axis=0, keepdims=True)
    masked = jnp.where(iota == idx1, neg_inf, lt)
    m2 = jnp.max(masked, axis=0, keepdims=True)
    idx2 = jnp.min(jnp.where(masked == m2, iota, E), axis=0, keepdims=True)

    e2 = jnp.exp(m2 - m1)
    w1 = 1.0 / (1.0 + e2)
    w2 = e2 * w1

    w_t = jnp.where(iota == idx1, w1, jnp.where(iota == idx2, w2,
                                                jnp.float32(0.0)))
    w_out_ref[...] = jnp.transpose(w_t)

    idx_t = jnp.where(iota == 0, idx1, jnp.where(iota == 1, idx2, 0))
    idx_out_ref[...] = jnp.transpose(idx_t)[:, :K]


@jax.jit
def kernel(x, W, noise_weight, noise):
    n = B * S
    x2 = x.reshape(n, D)
    wt = W.T
    nw = noise_weight.reshape(E, 1)
    noise2 = noise.reshape(n, E)

    grid = (n // T,)
    weights, idx = pl.pallas_call(
        _gate_body,
        grid=grid,
        in_specs=[
            pl.BlockSpec((T, D), lambda i: (i, 0)),
            pl.BlockSpec((D, E), lambda i: (0, 0)),
            pl.BlockSpec((E, 1), lambda i: (0, 0)),
            pl.BlockSpec((T, E), lambda i: (i, 0)),
        ],
        out_specs=[
            pl.BlockSpec((T, E), lambda i: (i, 0)),
            pl.BlockSpec((T, K), lambda i: (i, 0)),
        ],
        out_shape=[
            jax.ShapeDtypeStruct((n, E), jnp.float32),
            jax.ShapeDtypeStruct((n, K), jnp.int32),
        ],
        compiler_params=pltpu.CompilerParams(
            dimension_semantics=("arbitrary",),
        ),
    )(x2, wt, nw, noise2)

    return weights.reshape(B, S, E), idx.reshape(B, S, K)
